# baseline (speedup 1.0000x reference)
from math import sqrt

import jax
import jax.numpy as jnp
from jax import lax
from jax.experimental import pallas as pl
from jax.experimental.pallas import tpu as pltpu

_LOG2E = 1.4426950408889634
_QROWS = 136


def _row_tile(n, max_tile=512):
    if n <= max_tile:
        return n
    for t in (512, 256, 128, 64, 32, 16, 8):
        if n % t == 0:
            return t
    return n


def _make_setup_kernel(n_heads, d_keys):
    def _setup_kernel(src_ref, val_ref, wk_ref, bk_ref, wv_ref, bv_ref,
                      wqb_ref, m_ref, v_ref):
        src = src_ref[...].astype(jnp.bfloat16)
        val = val_ref[...].astype(jnp.bfloat16)
        k = jnp.dot(src, wk_ref[...], preferred_element_type=jnp.float32)
        v = jnp.dot(val, wv_ref[...], preferred_element_type=jnp.float32)
        k16 = (k + bk_ref[...]).astype(jnp.bfloat16)
        v16 = (v + bv_ref[...]).astype(jnp.bfloat16)

        m_parts = []
        for h in range(n_heads):
            sl = slice(h * d_keys, (h + 1) * d_keys)
            wqb_h = wqb_ref[h * _QROWS:(h + 1) * _QROWS, :]
            m_h = lax.dot_general(wqb_h, k16[:, sl], (((1,), (1,)), ((), ())),
                                  preferred_element_type=jnp.float32)
            m_parts.append(m_h.astype(jnp.bfloat16))
        m_ref[...] = jnp.concatenate(m_parts, axis=0)

        ts = v16.shape[0]
        lane = lax.broadcasted_iota(jnp.int32, (ts, d_keys), 1)
        ones_col = jnp.where(lane == 0, 1.0, 0.0).astype(jnp.bfloat16)
        pieces = []
        for h in range(n_heads):
            pieces.append(v16[:, h * d_keys:(h + 1) * d_keys])
            pieces.append(ones_col)
        v_ref[...] = jnp.concatenate(pieces, axis=-1)

    return _setup_kernel


def _make_attn_kernel(n_heads, d_keys, d_model):
    def _attn_kernel(x_ref, m_ref, v_ref, wo_ref, bo_ref, o_ref):
        tm = x_ref.shape[0]
        x16 = x_ref[...].astype(jnp.bfloat16)
        lane = lax.broadcasted_iota(jnp.int32, (tm, _QROWS - d_model), 1)
        ones_col = jnp.where(lane == 0, 1.0, 0.0).astype(jnp.bfloat16)
        x_aug = jnp.concatenate([x16, ones_col], axis=-1)

        m = m_ref[...]
        v = v_ref[...]

        def _score(h):
            m_h = m[h * _QROWS:(h + 1) * _QROWS, :]
            return lax.dot_general(x_aug, m_h, (((1,), (0,)), ((), ())),
                                   preferred_element_type=jnp.float32)

        s = _score(0)
        parts_t = []
        for h in range(n_heads):
            s_next = _score(h + 1) if h + 1 < n_heads else None
            p16 = jnp.exp2(jnp.minimum(s, 100.0)).astype(jnp.bfloat16)
            az_t = lax.dot_general(
                v[:, 2 * d_keys * h:2 * d_keys * (h + 1)], p16,
                (((0,), (1,)), ((), ())),
                preferred_element_type=jnp.float32)
            attn_t = az_t[:d_keys] * pl.reciprocal(
                az_t[d_keys:d_keys + 1], approx=True)
            parts_t.append(attn_t.astype(jnp.bfloat16))
            s = s_next

        a_t = jnp.concatenate(parts_t, axis=0)
        y = lax.dot_general(a_t, wo_ref[...], (((0,), (0,)), ((), ())),
                            preferred_element_type=jnp.float32)
        o_ref[...] = y + bo_ref[...]

    return _attn_kernel


def kernel(wq, bq, wk, bk, wv, bv, wo, bo,
           target_embedding, source_embedding, value_embedding):
    B, L, d_model = target_embedding.shape
    S, d_llm = source_embedding.shape
    HE = wq.shape[1]
    n_heads = 8
    E = HE // n_heads
    scale = 1.0 / sqrt(E)

    wk16 = wk.astype(jnp.bfloat16)
    wv16 = wv.astype(jnp.bfloat16)
    wo16 = wo.astype(jnp.bfloat16)

    c = scale * _LOG2E
    pad = jnp.zeros((_QROWS - d_model - 1, E), jnp.float32)
    wqb = jnp.concatenate(
        [jnp.concatenate([wq[:, h * E:(h + 1) * E] * c,
                          bq[:, h * E:(h + 1) * E] * c, pad], axis=0)
         for h in range(n_heads)], axis=0).astype(jnp.bfloat16)

    ts = _row_tile(S)
    const2 = lambda i: (0, 0)
    mq, v16 = pl.pallas_call(
        _make_setup_kernel(n_heads, E),
        out_shape=(jax.ShapeDtypeStruct((n_heads * _QROWS, S), jnp.bfloat16),
                   jax.ShapeDtypeStruct((S, 2 * HE), jnp.bfloat16)),
        grid=(pl.cdiv(S, ts),),
        in_specs=[
            pl.BlockSpec((ts, d_llm), lambda i: (i, 0)),
            pl.BlockSpec((ts, d_llm), lambda i: (i, 0)),
            pl.BlockSpec((d_llm, HE), const2),
            pl.BlockSpec((1, HE), const2),
            pl.BlockSpec((d_llm, HE), const2),
            pl.BlockSpec((1, HE), const2),
            pl.BlockSpec((n_heads * _QROWS, E), const2),
        ],
        out_specs=[pl.BlockSpec((n_heads * _QROWS, ts), lambda i: (0, i)),
                   pl.BlockSpec((ts, 2 * HE), lambda i: (i, 0))],
        compiler_params=pltpu.CompilerParams(
            dimension_semantics=("arbitrary",),
        ),
    )(source_embedding, value_embedding, wk16, bk, wv16, bv, wqb)

    BL = B * L
    x = target_embedding.reshape(BL, d_model)
    tm = _row_tile(BL)
    out = pl.pallas_call(
        _make_attn_kernel(n_heads, E, d_model),
        out_shape=jax.ShapeDtypeStruct((BL, d_llm), target_embedding.dtype),
        grid=(pl.cdiv(BL, tm),),
        in_specs=[
            pl.BlockSpec((tm, d_model), lambda i: (i, 0)),
            pl.BlockSpec((n_heads * _QROWS, S), const2),
            pl.BlockSpec((S, 2 * HE), const2),
            pl.BlockSpec((HE, d_llm), const2),
            pl.BlockSpec((1, d_llm), const2),
        ],
        out_specs=pl.BlockSpec((tm, d_llm), lambda i: (i, 0)),
        compiler_params=pltpu.CompilerParams(
            dimension_semantics=("arbitrary",),
        ),
    )(x, mq, v16, wo16, bo)

    return out.reshape(B, L, d_llm)


# speedup vs baseline: 1.1049x; 1.1049x over previous
from math import sqrt

import jax
import jax.numpy as jnp
from jax import lax
from jax.experimental import pallas as pl
from jax.experimental.pallas import tpu as pltpu

_LOG2E = 1.4426950408889634


def _row_tile(n, max_tile=512):
    if n <= max_tile:
        return n
    for t in (1024, 512, 256, 128, 64, 32, 16, 8):
        if t <= max_tile and n % t == 0:
            return t
    return n


def _make_kv_kernel(n_heads, d_keys):
    def _kv_kernel(src_ref, val_ref, wk_ref, bk_ref, wv_ref, bv_ref,
                   k_ref, v_ref):
        src = src_ref[...].astype(jnp.bfloat16)
        val = val_ref[...].astype(jnp.bfloat16)
        k = jnp.dot(src, wk_ref[...], preferred_element_type=jnp.float32)
        v = jnp.dot(val, wv_ref[...], preferred_element_type=jnp.float32)
        k_ref[...] = (k + bk_ref[...]).astype(jnp.bfloat16).T
        v16 = (v + bv_ref[...]).astype(jnp.bfloat16)
        ts = v16.shape[0]
        lane = lax.broadcasted_iota(jnp.int32, (ts, d_keys), 1)
        ones_col = jnp.where(lane == 0, 1.0, 0.0).astype(jnp.bfloat16)
        pieces = []
        for h in range(n_heads):
            pieces.append(v16[:, h * d_keys:(h + 1) * d_keys])
            pieces.append(ones_col)
        v_ref[...] = jnp.concatenate(pieces, axis=-1)

    return _kv_kernel


def _make_attn_kernel(n_heads, d_keys, scale):
    def _attn_kernel(x_ref, k_ref, v_ref, wq_ref, bq_ref, wo_ref, bo_ref,
                     o_ref):
        x = x_ref[...].astype(jnp.bfloat16)
        q = jnp.dot(x, wq_ref[...], preferred_element_type=jnp.float32)
        q = (q + bq_ref[...]) * (scale * _LOG2E)

        k = k_ref[...]
        v = v_ref[...]

        def _score(h):
            sl = slice(h * d_keys, (h + 1) * d_keys)
            q_h = q[:, sl].astype(jnp.bfloat16)
            return lax.dot_general(q_h, k[sl, :], (((1,), (0,)), ((), ())),
                                   preferred_element_type=jnp.float32)

        s = _score(0)
        parts_t = []
        for h in range(n_heads):
            s_next = _score(h + 1) if h + 1 < n_heads else None
            p16 = jnp.exp2(jnp.minimum(s, 100.0)).astype(jnp.bfloat16)
            az_t = lax.dot_general(
                v[:, 2 * d_keys * h:2 * d_keys * (h + 1)], p16,
                (((0,), (1,)), ((), ())),
                preferred_element_type=jnp.float32)
            attn_t = az_t[:d_keys] * pl.reciprocal(
                az_t[d_keys:d_keys + 1], approx=True)
            parts_t.append(attn_t.astype(jnp.bfloat16))
            s = s_next

        a_t = jnp.concatenate(parts_t, axis=0)
        y = lax.dot_general(a_t, wo_ref[...], (((0,), (0,)), ((), ())),
                            preferred_element_type=jnp.float32)
        o_ref[...] = y + bo_ref[...]

    return _attn_kernel


def kernel(wq, bq, wk, bk, wv, bv, wo, bo,
           target_embedding, source_embedding, value_embedding):
    B, L, d_model = target_embedding.shape
    S, d_llm = source_embedding.shape
    HE = wq.shape[1]
    n_heads = 8
    E = HE // n_heads
    scale = 1.0 / sqrt(E)

    wq16 = wq.astype(jnp.bfloat16)
    wk16 = wk.astype(jnp.bfloat16)
    wv16 = wv.astype(jnp.bfloat16)
    wo16 = wo.astype(jnp.bfloat16)

    ts = _row_tile(S)
    const2 = lambda i: (0, 0)
    k16, v16 = pl.pallas_call(
        _make_kv_kernel(n_heads, E),
        out_shape=(jax.ShapeDtypeStruct((HE, S), jnp.bfloat16),
                   jax.ShapeDtypeStruct((S, 2 * HE), jnp.bfloat16)),
        grid=(pl.cdiv(S, ts),),
        in_specs=[
            pl.BlockSpec((ts, d_llm), lambda i: (i, 0)),
            pl.BlockSpec((ts, d_llm), lambda i: (i, 0)),
            pl.BlockSpec((d_llm, HE), const2),
            pl.BlockSpec((1, HE), const2),
            pl.BlockSpec((d_llm, HE), const2),
            pl.BlockSpec((1, HE), const2),
        ],
        out_specs=[pl.BlockSpec((HE, ts), lambda i: (0, i)),
                   pl.BlockSpec((ts, 2 * HE), lambda i: (i, 0))],
        compiler_params=pltpu.CompilerParams(
            dimension_semantics=("arbitrary",),
        ),
    )(source_embedding, value_embedding, wk16, bk, wv16, bv)

    BL = B * L
    x = target_embedding.reshape(BL, d_model)
    tm = _row_tile(BL, max_tile=1024)
    out = pl.pallas_call(
        _make_attn_kernel(n_heads, E, scale),
        out_shape=jax.ShapeDtypeStruct((BL, d_llm), target_embedding.dtype),
        grid=(pl.cdiv(BL, tm),),
        in_specs=[
            pl.BlockSpec((tm, d_model), lambda i: (i, 0)),
            pl.BlockSpec((HE, S), const2),
            pl.BlockSpec((S, 2 * HE), const2),
            pl.BlockSpec((d_model, HE), const2),
            pl.BlockSpec((1, HE), const2),
            pl.BlockSpec((HE, d_llm), const2),
            pl.BlockSpec((1, d_llm), const2),
        ],
        out_specs=pl.BlockSpec((tm, d_llm), lambda i: (i, 0)),
        compiler_params=pltpu.CompilerParams(
            dimension_semantics=("arbitrary",),
        ),
    )(x, k16, v16, wq16, bq, wo16, bo)

    return out.reshape(B, L, d_llm)


# speedup vs baseline: 1.1714x; 1.0602x over previous
from math import sqrt

import jax
import jax.numpy as jnp
from jax import lax
from jax.experimental import pallas as pl
from jax.experimental.pallas import tpu as pltpu

_LOG2E = 1.4426950408889634


def _row_tile(n, max_tile=1024):
    if n <= max_tile:
        return n
    for t in (1024, 512, 256, 128, 64, 32, 16, 8):
        if t <= max_tile and n % t == 0:
            return t
    return n


def _make_fused_kernel(n_heads, d_keys, scale):
    def _fused(x_ref, src_ref, val_ref, wq_ref, bq_ref, wk_ref, bk_ref,
               wv_ref, bv_ref, wo_ref, bo_ref, o_ref,
               k_scr, v_scr, wq_scr, wo_scr):
        i = pl.program_id(0)

        @pl.when(i == 0)
        def _setup():
            src = src_ref[...].astype(jnp.bfloat16)
            val = val_ref[...].astype(jnp.bfloat16)
            wk16 = wk_ref[...].astype(jnp.bfloat16)
            wv16 = wv_ref[...].astype(jnp.bfloat16)
            k = jnp.dot(src, wk16, preferred_element_type=jnp.float32)
            v = jnp.dot(val, wv16, preferred_element_type=jnp.float32)
            k_scr[...] = (k + bk_ref[...]).astype(jnp.bfloat16).T
            v16 = (v + bv_ref[...]).astype(jnp.bfloat16)
            ts = v16.shape[0]
            lane = lax.broadcasted_iota(jnp.int32, (ts, d_keys), 1)
            ones_col = jnp.where(lane == 0, 1.0, 0.0).astype(jnp.bfloat16)
            pieces = []
            for h in range(n_heads):
                pieces.append(v16[:, h * d_keys:(h + 1) * d_keys])
                pieces.append(ones_col)
            v_scr[...] = jnp.concatenate(pieces, axis=-1)
            wq_scr[...] = wq_ref[...].astype(jnp.bfloat16)
            wo_scr[...] = wo_ref[...].astype(jnp.bfloat16)

        @pl.when(i > 0)
        def _attn():
            x = x_ref[...].astype(jnp.bfloat16)
            q = jnp.dot(x, wq_scr[...], preferred_element_type=jnp.float32)
            q = (q + bq_ref[...]) * (scale * _LOG2E)

            k = k_scr[...]
            v = v_scr[...]

            def _score(h):
                sl = slice(h * d_keys, (h + 1) * d_keys)
                q_h = q[:, sl].astype(jnp.bfloat16)
                return lax.dot_general(q_h, k[sl, :],
                                       (((1,), (0,)), ((), ())),
                                       preferred_element_type=jnp.float32)

            s = _score(0)
            parts_t = []
            for h in range(n_heads):
                s_next = _score(h + 1) if h + 1 < n_heads else None
                p16 = jnp.exp2(jnp.minimum(s, 100.0)).astype(jnp.bfloat16)
                az_t = lax.dot_general(
                    v[:, 2 * d_keys * h:2 * d_keys * (h + 1)], p16,
                    (((0,), (1,)), ((), ())),
                    preferred_element_type=jnp.float32)
                attn_t = az_t[:d_keys] * pl.reciprocal(
                    az_t[d_keys:d_keys + 1], approx=True)
                parts_t.append(attn_t.astype(jnp.bfloat16))
                s = s_next

            a_t = jnp.concatenate(parts_t, axis=0)
            y = lax.dot_general(a_t, wo_scr[...], (((0,), (0,)), ((), ())),
                                preferred_element_type=jnp.float32)
            o_ref[...] = y + bo_ref[...]

    return _fused


def kernel(wq, bq, wk, bk, wv, bv, wo, bo,
           target_embedding, source_embedding, value_embedding):
    B, L, d_model = target_embedding.shape
    S, d_llm = source_embedding.shape
    HE = wq.shape[1]
    n_heads = 8
    E = HE // n_heads
    scale = 1.0 / sqrt(E)

    BL = B * L
    x = target_embedding.reshape(BL, d_model)
    tm = _row_tile(BL)
    n_tiles = pl.cdiv(BL, tm)

    const2 = lambda i: (0, 0)
    prev = lambda i: (jnp.maximum(i - 1, 0), 0)
    out = pl.pallas_call(
        _make_fused_kernel(n_heads, E, scale),
        out_shape=jax.ShapeDtypeStruct((BL, d_llm), target_embedding.dtype),
        grid=(n_tiles + 1,),
        in_specs=[
            pl.BlockSpec((tm, d_model), prev),
            pl.BlockSpec((S, d_llm), const2),
            pl.BlockSpec((S, d_llm), const2),
            pl.BlockSpec((d_model, HE), const2),
            pl.BlockSpec((1, HE), const2),
            pl.BlockSpec((d_llm, HE), const2),
            pl.BlockSpec((1, HE), const2),
            pl.BlockSpec((d_llm, HE), const2),
            pl.BlockSpec((1, HE), const2),
            pl.BlockSpec((HE, d_llm), const2),
            pl.BlockSpec((1, d_llm), const2),
        ],
        out_specs=pl.BlockSpec((tm, d_llm), prev),
        scratch_shapes=[
            pltpu.VMEM((HE, S), jnp.bfloat16),
            pltpu.VMEM((S, 2 * HE), jnp.bfloat16),
            pltpu.VMEM((d_model, HE), jnp.bfloat16),
            pltpu.VMEM((HE, d_llm), jnp.bfloat16),
        ],
        compiler_params=pltpu.CompilerParams(
            dimension_semantics=("arbitrary",),
        ),
    )(x, source_embedding, value_embedding, wq, bq, wk, bk, wv, bv, wo, bo)

    return out.reshape(B, L, d_llm)


# speedup vs baseline: 1.1796x; 1.0070x over previous
from math import sqrt

import jax
import jax.numpy as jnp
from jax import lax
from jax.experimental import pallas as pl
from jax.experimental.pallas import tpu as pltpu

_LOG2E = 1.4426950408889634


def _row_tile(n, max_tile=1024):
    if n <= max_tile:
        return n
    for t in (1024, 512, 256, 128, 64, 32, 16, 8):
        if t <= max_tile and n % t == 0:
            return t
    return n


def _make_fused_kernel(n_heads, d_keys, scale):
    def _fused(x_ref, src_ref, val_ref, wq_ref, bq_ref, wk_ref, bk_ref,
               wv_ref, bv_ref, wo_ref, bo_ref, o_ref,
               k_scr, v_scr, wq_scr, wo_scr):
        i = pl.program_id(0)

        @pl.when(i == 0)
        def _setup():
            src = src_ref[...].astype(jnp.bfloat16)
            val = val_ref[...].astype(jnp.bfloat16)
            wk16 = wk_ref[...].astype(jnp.bfloat16)
            wv16 = wv_ref[...].astype(jnp.bfloat16)
            k = jnp.dot(src, wk16, preferred_element_type=jnp.float32)
            v = jnp.dot(val, wv16, preferred_element_type=jnp.float32)
            k_scr[...] = (k + bk_ref[...]).astype(jnp.bfloat16).T
            v16 = (v + bv_ref[...]).astype(jnp.bfloat16)
            ts = v16.shape[0]
            lane = lax.broadcasted_iota(jnp.int32, (ts, d_keys), 1)
            ones_col = jnp.where(lane == 0, 1.0, 0.0).astype(jnp.bfloat16)
            pieces = []
            for h in range(n_heads):
                pieces.append(v16[:, h * d_keys:(h + 1) * d_keys])
                pieces.append(ones_col)
            v_scr[...] = jnp.concatenate(pieces, axis=-1)
            wq_scr[...] = (wq_ref[...] * (scale * _LOG2E)).astype(jnp.bfloat16)
            wo_scr[...] = wo_ref[...].astype(jnp.bfloat16)

        @pl.when(i > 0)
        def _attn():
            x = x_ref[...].astype(jnp.bfloat16)
            q = jnp.dot(x, wq_scr[...], preferred_element_type=jnp.float32)
            q = q + bq_ref[...] * (scale * _LOG2E)

            k = k_scr[...]
            v = v_scr[...]

            def _score(h):
                sl = slice(h * d_keys, (h + 1) * d_keys)
                q_h = q[:, sl].astype(jnp.bfloat16)
                return lax.dot_general(q_h, k[sl, :],
                                       (((1,), (0,)), ((), ())),
                                       preferred_element_type=jnp.float32)

            s = _score(0)
            parts_t = []
            for h in range(n_heads):
                s_next = _score(h + 1) if h + 1 < n_heads else None
                p16 = jnp.exp2(jnp.minimum(s, 100.0)).astype(jnp.bfloat16)
                az_t = lax.dot_general(
                    v[:, 2 * d_keys * h:2 * d_keys * (h + 1)], p16,
                    (((0,), (1,)), ((), ())),
                    preferred_element_type=jnp.float32)
                attn_t = az_t[:d_keys] * pl.reciprocal(
                    az_t[d_keys:d_keys + 1], approx=True)
                parts_t.append(attn_t.astype(jnp.bfloat16))
                s = s_next

            a_t = jnp.concatenate(parts_t, axis=0)
            y = lax.dot_general(a_t, wo_scr[...], (((0,), (0,)), ((), ())),
                                preferred_element_type=jnp.float32)
            o_ref[...] = y + bo_ref[...]

    return _fused


def kernel(wq, bq, wk, bk, wv, bv, wo, bo,
           target_embedding, source_embedding, value_embedding):
    B, L, d_model = target_embedding.shape
    S, d_llm = source_embedding.shape
    HE = wq.shape[1]
    n_heads = 8
    E = HE // n_heads
    scale = 1.0 / sqrt(E)

    BL = B * L
    x = target_embedding.reshape(BL, d_model)
    tm = _row_tile(BL)
    n_tiles = pl.cdiv(BL, tm)

    const2 = lambda i: (0, 0)
    prev = lambda i: (jnp.maximum(i - 1, 0), 0)
    out = pl.pallas_call(
        _make_fused_kernel(n_heads, E, scale),
        out_shape=jax.ShapeDtypeStruct((BL, d_llm), target_embedding.dtype),
        grid=(n_tiles + 1,),
        in_specs=[
            pl.BlockSpec((tm, d_model), prev),
            pl.BlockSpec((S, d_llm), const2),
            pl.BlockSpec((S, d_llm), const2),
            pl.BlockSpec((d_model, HE), const2),
            pl.BlockSpec((1, HE), const2),
            pl.BlockSpec((d_llm, HE), const2),
            pl.BlockSpec((1, HE), const2),
            pl.BlockSpec((d_llm, HE), const2),
            pl.BlockSpec((1, HE), const2),
            pl.BlockSpec((HE, d_llm), const2),
            pl.BlockSpec((1, d_llm), const2),
        ],
        out_specs=pl.BlockSpec((tm, d_llm), prev),
        scratch_shapes=[
            pltpu.VMEM((HE, S), jnp.bfloat16),
            pltpu.VMEM((S, 2 * HE), jnp.bfloat16),
            pltpu.VMEM((d_model, HE), jnp.bfloat16),
            pltpu.VMEM((HE, d_llm), jnp.bfloat16),
        ],
        compiler_params=pltpu.CompilerParams(
            dimension_semantics=("arbitrary",),
        ),
    )(x, source_embedding, value_embedding, wq, bq, wk, bk, wv, bv, wo, bo)

    return out.reshape(B, L, d_llm)
